# Initial kernel scaffold; baseline (speedup 1.0000x reference)
import jax
import jax.numpy as jnp
from jax.experimental import pallas as pl


def kernel(input_waves, durations, durations_lengths):
    raise NotImplementedError("write your pallas kernel here")


# speedup vs baseline: 82.9677x; 82.9677x over previous
import numpy as np
import jax
import jax.numpy as jnp
from jax.experimental import pallas as pl

_N_FFT = 1024
_HOP = 256
_PAD = _N_FFT // 2

_n = np.arange(_N_FFT)
_w = (0.5 - 0.5 * np.cos(2.0 * np.pi * _n / _N_FFT)).astype(np.float32)
_walt = (_w * np.where(_n % 2 == 0, 1.0, -1.0)).astype(np.float32)
_wsq = (_w * _w).astype(np.float32)
_WMAT = np.zeros((16, _HOP), np.float32)
_WMAT[0:4] = _w.reshape(4, _HOP)
_WMAT[4:8] = _walt.reshape(4, _HOP)
_WMAT[8:12] = _wsq.reshape(4, _HOP)


def _energy_tok_kernel(x_ref, w_ref, d_ref, out_ref):
    x = x_ref[:]
    w = w_ref[:]
    d = d_ref[:]
    B, n_chunks, _ = x.shape
    n_frames = n_chunks - 3
    n_tok = d.shape[1]

    xsq = x * x
    a = []
    for j in range(4):
        wj = w[j:j + 1, :].reshape(1, 1, _HOP)
        wjalt = w[j + 4:j + 5, :].reshape(1, 1, _HOP)
        wjsq = w[j + 8:j + 9, :].reshape(1, 1, _HOP)
        a.append((jnp.sum(x * wj, axis=-1),
                  jnp.sum(x * wjalt, axis=-1),
                  jnp.sum(xsq * wjsq, axis=-1)))

    s0 = a[0][0][:, 0:n_frames] + a[1][0][:, 1:n_frames + 1] \
        + a[2][0][:, 2:n_frames + 2] + a[3][0][:, 3:n_frames + 3]
    s1 = a[0][1][:, 0:n_frames] + a[1][1][:, 1:n_frames + 1] \
        + a[2][1][:, 2:n_frames + 2] + a[3][1][:, 3:n_frames + 3]
    s2 = a[0][2][:, 0:n_frames] + a[1][2][:, 1:n_frames + 1] \
        + a[2][2][:, 2:n_frames + 2] + a[3][2][:, 3:n_frames + 3]

    power = 0.5 * (_N_FFT * s2 + s0 * s0 + s1 * s1)
    e = jnp.sqrt(jnp.maximum(power, 1e-10))

    df = d.astype(jnp.float32)
    iu = jax.lax.broadcasted_iota(jnp.int32, (n_tok, n_tok), 0)
    it = jax.lax.broadcasted_iota(jnp.int32, (n_tok, n_tok), 1)
    tri = (iu <= it).astype(jnp.float32)
    ends = jax.lax.dot_general(df, tri, (((1,), (0,)), ((), ())),
                               preferred_element_type=jnp.float32)
    starts = ends - df

    f = jax.lax.broadcasted_iota(
        jnp.int32, (B, n_tok, n_frames), 2).astype(jnp.float32)
    mask = (f >= starts[:, :, None]) & (f < ends[:, :, None])
    seg = jnp.sum(jnp.where(mask, e[:, None, :], 0.0), axis=-1)
    cnt = jnp.sum(mask.astype(jnp.float32), axis=-1)
    tok = jnp.where(cnt > 0, seg / jnp.maximum(cnt, 1.0), 0.0)

    e0 = tok[0:1, :]
    m0 = e0 != 0.0
    cnt0 = jnp.maximum(jnp.sum(m0.astype(jnp.float32)), 1.0)
    avg = jnp.sum(jnp.where(m0, e0, 0.0)) / cnt0
    out_ref[:] = tok / avg


def kernel(input_waves, durations, durations_lengths):
    B, T = input_waves.shape
    n_chunks = (T + 2 * _PAD) // _HOP
    xp = jnp.pad(input_waves, ((0, 0), (_PAD, _PAD)), mode='reflect')
    x3 = xp.reshape(B, n_chunks, _HOP)
    d = durations.astype(jnp.int32)
    tok = pl.pallas_call(
        _energy_tok_kernel,
        out_shape=jax.ShapeDtypeStruct((B, durations.shape[1]), jnp.float32),
    )(x3, jnp.asarray(_WMAT), d)
    return (tok[..., None], durations_lengths)
